# baseline (speedup 1.0000x reference)
import functools

import jax
import jax.numpy as jnp
from jax.experimental import pallas as pl
from jax.experimental.pallas import tpu as pltpu
from jax.experimental.pallas import tpu_sc as plsc

EMB_DIM = 64
NUM_CODES = 1024
N_ROWS = 16 * 1024
N_CHUNKS = 1
CHUNK_ROWS = N_ROWS // N_CHUNKS
ROW_TILE = 2048
GATHER_WINDOW = 128
GATHER_WIDTH = 128


def _vq_tc_body(n_tiles, z_ref, cb_ref, zsq_ref, csq_ref, idx_ref, loss_ref):
    z = z_ref[...]
    cb = cb_ref[...]
    dots = jax.lax.dot_general(z, cb, (((1,), (1,)), ((), ())),
                               preferred_element_type=jnp.float32)
    d2 = zsq_ref[...] + csq_ref[...] - 2.0 * dots
    dist = jnp.sqrt(jnp.clip(d2, 0.0, None))
    dmin = jnp.min(dist, axis=1, keepdims=True)
    ids = jax.lax.broadcasted_iota(jnp.int32, dist.shape, 1)
    idx_ref[...] = jnp.min(jnp.where(dist == dmin, ids, NUM_CODES), axis=1)
    part = jnp.sum(dmin * dmin, keepdims=True)

    @pl.when(pl.program_id(0) == 0)
    def _init():
        loss_ref[...] = jnp.zeros_like(part)

    loss_ref[...] += part


def _vq_distances_argmin(z_chunk, codebook, csq):
    n_rows = z_chunk.shape[0]
    n_tiles = n_rows // ROW_TILE
    zsq = jnp.sum(z_chunk ** 2, axis=1, keepdims=True)
    return pl.pallas_call(
        functools.partial(_vq_tc_body, n_tiles),
        grid=(n_tiles,),
        in_specs=[
            pl.BlockSpec((ROW_TILE, EMB_DIM), lambda i: (i, 0)),
            pl.BlockSpec((NUM_CODES, EMB_DIM), lambda i: (0, 0)),
            pl.BlockSpec((ROW_TILE, 1), lambda i: (i, 0)),
            pl.BlockSpec((1, NUM_CODES), lambda i: (0, 0)),
        ],
        out_specs=[
            pl.BlockSpec((ROW_TILE,), lambda i: (i,)),
            pl.BlockSpec((1, 1), lambda i: (0, 0)),
        ],
        out_shape=[
            jax.ShapeDtypeStruct((n_rows,), jnp.int32),
            jax.ShapeDtypeStruct((1, 1), jnp.float32),
        ],
    )(z_chunk, codebook, zsq, csq)


def _sc_gather(codebook_padded, indices_2d):
    n_idx = indices_2d.shape[1]
    mesh = plsc.VectorSubcoreMesh(core_axis_name="core",
                                  subcore_axis_name="subcore")

    @pl.kernel(out_type=jax.ShapeDtypeStruct((n_idx, GATHER_WIDTH),
                                             jnp.float32),
               mesh=mesh)
    def gather_kernel(cb_hbm, i_hbm, o_hbm):
        def body(i_vmem, o_vmem):
            pltpu.sync_copy(cb_hbm.at[i_vmem.at[0]], o_vmem)

        pltpu.emit_pipeline(
            body,
            grid=(n_idx // GATHER_WINDOW,),
            in_specs=[pl.BlockSpec((1, GATHER_WINDOW),
                                   index_map=lambda i: (0, i))],
            out_specs=[pl.BlockSpec((GATHER_WINDOW, GATHER_WIDTH),
                                    index_map=lambda i: (i, 0))],
            core_axis_name=("core", "subcore"),
            dimension_semantics=(pltpu.PARALLEL,),
        )(i_hbm, o_hbm)

    return gather_kernel(codebook_padded, indices_2d)


def kernel(z, codebook):
    z_flat = z.reshape(-1, EMB_DIM)
    csq = jnp.sum(codebook ** 2, axis=1)[None, :]
    cb_padded = jnp.pad(codebook, ((0, 0), (0, GATHER_WIDTH - EMB_DIM)))

    idx_chunks, loss_parts, zq_chunks = [], [], []
    for k in range(N_CHUNKS):
        z_chunk = jax.lax.slice_in_dim(z_flat, k * CHUNK_ROWS,
                                       (k + 1) * CHUNK_ROWS, axis=0)
        idx1d, lpart = _vq_distances_argmin(z_chunk, codebook, csq)
        idx_chunks.append(idx1d)
        loss_parts.append(lpart)
        zq_chunks.append(_sc_gather(cb_padded, idx1d.reshape(1, CHUNK_ROWS)))

    loss = (sum(p[0, 0] for p in loss_parts) * (1.25 / (N_ROWS * EMB_DIM)))
    encoding_indices = jnp.concatenate(idx_chunks, axis=0).reshape(N_ROWS)
    z_q = jnp.concatenate([c[:, :EMB_DIM] for c in zq_chunks], axis=0)
    return z_q.reshape(z.shape), loss, encoding_indices


# speedup vs baseline: 1.2203x; 1.2203x over previous
import jax
import jax.numpy as jnp
from jax.experimental import pallas as pl
from jax.experimental.pallas import tpu as pltpu
from jax.experimental.pallas import tpu_sc as plsc

EMB_DIM = 64
NUM_CODES = 1024
BATCH = 16
POSITIONS = 1024
N_ROWS = BATCH * POSITIONS
GATHER_WINDOW = 128
GATHER_WIDTH = 128


def _vq_tc_body(zt_ref, cb_ref, zsq_ref, csq_ref, idx_ref, loss_ref):
    zt = zt_ref[0]
    cb = cb_ref[...]
    dotsT = jax.lax.dot_general(cb, zt, (((1,), (0,)), ((), ())),
                                preferred_element_type=jnp.float32)
    d2T = zsq_ref[0] + csq_ref[...] - 2.0 * dotsT
    distT = jnp.sqrt(jnp.clip(d2T, 0.0, None))
    dminT = jnp.min(distT, axis=0, keepdims=True)
    ids = jax.lax.broadcasted_iota(jnp.int32, distT.shape, 0)
    idx_ref[0] = jnp.min(jnp.where(distT == dminT, ids, NUM_CODES), axis=0,
                         keepdims=True)
    part = jnp.sum(dminT * dminT, keepdims=True)

    @pl.when(pl.program_id(0) == 0)
    def _init():
        loss_ref[...] = jnp.zeros_like(part)

    loss_ref[...] += part

    @pl.when(pl.program_id(0) == BATCH - 1)
    def _finish():
        loss_ref[...] = loss_ref[...] * (1.25 / (N_ROWS * EMB_DIM))


def _vq_distances_argmin(z, codebook):
    zt = z.transpose(0, 2, 1)
    z_flat = z.reshape(-1, EMB_DIM)
    zsq = jnp.sum(z_flat ** 2, axis=1, keepdims=True).reshape(BATCH, 1,
                                                              POSITIONS)
    csq = jnp.sum(codebook ** 2, axis=1)[:, None]
    return pl.pallas_call(
        _vq_tc_body,
        grid=(BATCH,),
        in_specs=[
            pl.BlockSpec((1, EMB_DIM, POSITIONS), lambda i: (i, 0, 0)),
            pl.BlockSpec((NUM_CODES, EMB_DIM), lambda i: (0, 0)),
            pl.BlockSpec((1, 1, POSITIONS), lambda i: (i, 0, 0)),
            pl.BlockSpec((NUM_CODES, 1), lambda i: (0, 0)),
        ],
        out_specs=[
            pl.BlockSpec((1, 1, POSITIONS), lambda i: (i, 0, 0)),
            pl.BlockSpec((1, 1), lambda i: (0, 0)),
        ],
        out_shape=[
            jax.ShapeDtypeStruct((BATCH, 1, POSITIONS), jnp.int32),
            jax.ShapeDtypeStruct((1, 1), jnp.float32),
        ],
    )(zt, codebook, zsq, csq)


def _sc_gather(codebook_padded, indices_2d):
    n_idx = indices_2d.shape[1]
    mesh = plsc.VectorSubcoreMesh(core_axis_name="core",
                                  subcore_axis_name="subcore")

    @pl.kernel(out_type=jax.ShapeDtypeStruct((n_idx, GATHER_WIDTH),
                                             jnp.float32),
               mesh=mesh)
    def gather_kernel(cb_hbm, i_hbm, o_hbm):
        def body(i_vmem, o_vmem):
            pltpu.sync_copy(cb_hbm.at[i_vmem.at[0]], o_vmem)

        pltpu.emit_pipeline(
            body,
            grid=(n_idx // GATHER_WINDOW,),
            in_specs=[pl.BlockSpec((1, GATHER_WINDOW),
                                   index_map=lambda i: (0, i))],
            out_specs=[pl.BlockSpec((GATHER_WINDOW, GATHER_WIDTH),
                                    index_map=lambda i: (i, 0))],
            core_axis_name=("core", "subcore"),
            dimension_semantics=(pltpu.PARALLEL,),
        )(i_hbm, o_hbm)

    return gather_kernel(codebook_padded, indices_2d)


def kernel(z, codebook):
    idx3, loss = _vq_distances_argmin(z, codebook)
    cb_padded = jnp.pad(codebook, ((0, 0), (0, GATHER_WIDTH - EMB_DIM)))
    z_q = _sc_gather(cb_padded, idx3.reshape(1, N_ROWS))
    encoding_indices = idx3.reshape(N_ROWS)
    return (z_q[:, :EMB_DIM].reshape(z.shape), loss.reshape(()),
            encoding_indices)


# speedup vs baseline: 1.2261x; 1.0047x over previous
import jax
import jax.numpy as jnp
from jax.experimental import pallas as pl
from jax.experimental.pallas import tpu as pltpu
from jax.experimental.pallas import tpu_sc as plsc

EMB_DIM = 64
NUM_CODES = 1024
BATCH = 16
POSITIONS = 1024
N_ROWS = BATCH * POSITIONS
GATHER_WINDOW = 256
GATHER_WIDTH = 128


def _vq_tc_body(zt_ref, cb_ref, zsq_ref, csq_ref, idx_ref, loss_ref):
    zt = zt_ref[0]
    cb = cb_ref[...]
    dotsT = jax.lax.dot_general(cb, zt, (((1,), (0,)), ((), ())),
                                preferred_element_type=jnp.float32)
    d2T = zsq_ref[0] + csq_ref[...] - 2.0 * dotsT
    distT = jnp.sqrt(jnp.clip(d2T, 0.0, None))
    dminT = jnp.min(distT, axis=0, keepdims=True)
    ids = jax.lax.broadcasted_iota(jnp.int32, distT.shape, 0)
    idx_ref[0] = jnp.min(jnp.where(distT == dminT, ids, NUM_CODES), axis=0,
                         keepdims=True)
    part = jnp.sum(dminT * dminT, keepdims=True)

    @pl.when(pl.program_id(0) == 0)
    def _init():
        loss_ref[...] = jnp.zeros_like(part)

    loss_ref[...] += part

    @pl.when(pl.program_id(0) == BATCH - 1)
    def _finish():
        loss_ref[...] = loss_ref[...] * (1.25 / (N_ROWS * EMB_DIM))


def _vq_distances_argmin(z, codebook):
    zt = z.transpose(0, 2, 1)
    z_flat = z.reshape(-1, EMB_DIM)
    zsq = jnp.sum(z_flat ** 2, axis=1, keepdims=True).reshape(BATCH, 1,
                                                              POSITIONS)
    csq = jnp.sum(codebook ** 2, axis=1)[:, None]
    return pl.pallas_call(
        _vq_tc_body,
        grid=(BATCH,),
        in_specs=[
            pl.BlockSpec((1, EMB_DIM, POSITIONS), lambda i: (i, 0, 0)),
            pl.BlockSpec((NUM_CODES, EMB_DIM), lambda i: (0, 0)),
            pl.BlockSpec((1, 1, POSITIONS), lambda i: (i, 0, 0)),
            pl.BlockSpec((NUM_CODES, 1), lambda i: (0, 0)),
        ],
        out_specs=[
            pl.BlockSpec((1, 1, POSITIONS), lambda i: (i, 0, 0)),
            pl.BlockSpec((1, 1), lambda i: (0, 0)),
        ],
        out_shape=[
            jax.ShapeDtypeStruct((BATCH, 1, POSITIONS), jnp.int32),
            jax.ShapeDtypeStruct((1, 1), jnp.float32),
        ],
    )(zt, codebook, zsq, csq)


def _sc_gather(codebook_padded, indices_2d):
    n_idx = indices_2d.shape[1]
    mesh = plsc.VectorSubcoreMesh(core_axis_name="core",
                                  subcore_axis_name="subcore")

    @pl.kernel(out_type=jax.ShapeDtypeStruct((n_idx, GATHER_WIDTH),
                                             jnp.float32),
               mesh=mesh)
    def gather_kernel(cb_hbm, i_hbm, o_hbm):
        def body(i_vmem, o_vmem):
            pltpu.sync_copy(cb_hbm.at[i_vmem.at[0]], o_vmem)

        pltpu.emit_pipeline(
            body,
            grid=(n_idx // GATHER_WINDOW,),
            in_specs=[pl.BlockSpec((1, GATHER_WINDOW),
                                   index_map=lambda i: (0, i))],
            out_specs=[pl.BlockSpec((GATHER_WINDOW, GATHER_WIDTH),
                                    index_map=lambda i: (i, 0))],
            core_axis_name=("core", "subcore"),
            dimension_semantics=(pltpu.PARALLEL,),
        )(i_hbm, o_hbm)

    return gather_kernel(codebook_padded, indices_2d)


def kernel(z, codebook):
    idx3, loss = _vq_distances_argmin(z, codebook)
    cb_padded = jnp.pad(codebook, ((0, 0), (0, GATHER_WIDTH - EMB_DIM)))
    z_q = _sc_gather(cb_padded, idx3.reshape(1, N_ROWS))
    encoding_indices = idx3.reshape(N_ROWS)
    return (z_q[:, :EMB_DIM].reshape(z.shape), loss.reshape(()),
            encoding_indices)


# speedup vs baseline: 1.2609x; 1.0284x over previous
import jax
import jax.numpy as jnp
from jax.experimental import pallas as pl
from jax.experimental.pallas import tpu as pltpu
from jax.experimental.pallas import tpu_sc as plsc

EMB_DIM = 64
NUM_CODES = 1024
BATCH = 16
POSITIONS = 1024
N_ROWS = BATCH * POSITIONS
GATHER_WINDOW = 256
GATHER_WIDTH = 128


BATCH_PER_TILE = 2
N_TILES = BATCH // BATCH_PER_TILE


def _vq_tc_body(zt_ref, cb_ref, zsq_ref, csq_ref, idx_ref, loss_ref):
    cb = cb_ref[...]
    part = None
    for k in range(BATCH_PER_TILE):
        zt = zt_ref[k]
        dotsT = jax.lax.dot_general(cb, zt, (((1,), (0,)), ((), ())),
                                    preferred_element_type=jnp.float32)
        d2T = zsq_ref[k] + csq_ref[...] - 2.0 * dotsT
        distT = jnp.sqrt(jnp.clip(d2T, 0.0, None))
        dminT = jnp.min(distT, axis=0, keepdims=True)
        ids = jax.lax.broadcasted_iota(jnp.int32, distT.shape, 0)
        idx_ref[k] = jnp.min(jnp.where(distT == dminT, ids, NUM_CODES),
                             axis=0, keepdims=True)
        p = jnp.sum(dminT * dminT, keepdims=True)
        part = p if part is None else part + p

    @pl.when(pl.program_id(0) == 0)
    def _init():
        loss_ref[...] = jnp.zeros_like(part)

    loss_ref[...] += part

    @pl.when(pl.program_id(0) == N_TILES - 1)
    def _finish():
        loss_ref[...] = loss_ref[...] * (1.25 / (N_ROWS * EMB_DIM))


def _vq_distances_argmin(z, codebook):
    zt = z.transpose(0, 2, 1)
    z_flat = z.reshape(-1, EMB_DIM)
    zsq = jnp.sum(z_flat ** 2, axis=1, keepdims=True).reshape(BATCH, 1,
                                                              POSITIONS)
    csq = jnp.sum(codebook ** 2, axis=1)[:, None]
    return pl.pallas_call(
        _vq_tc_body,
        grid=(N_TILES,),
        in_specs=[
            pl.BlockSpec((BATCH_PER_TILE, EMB_DIM, POSITIONS),
                         lambda i: (i, 0, 0)),
            pl.BlockSpec((NUM_CODES, EMB_DIM), lambda i: (0, 0)),
            pl.BlockSpec((BATCH_PER_TILE, 1, POSITIONS),
                         lambda i: (i, 0, 0)),
            pl.BlockSpec((NUM_CODES, 1), lambda i: (0, 0)),
        ],
        out_specs=[
            pl.BlockSpec((BATCH_PER_TILE, 1, POSITIONS),
                         lambda i: (i, 0, 0)),
            pl.BlockSpec((1, 1), lambda i: (0, 0)),
        ],
        out_shape=[
            jax.ShapeDtypeStruct((BATCH, 1, POSITIONS), jnp.int32),
            jax.ShapeDtypeStruct((1, 1), jnp.float32),
        ],
    )(zt, codebook, zsq, csq)


def _sc_gather(codebook_padded, indices_2d):
    n_idx = indices_2d.shape[1]
    mesh = plsc.VectorSubcoreMesh(core_axis_name="core",
                                  subcore_axis_name="subcore")

    @pl.kernel(out_type=jax.ShapeDtypeStruct((n_idx, GATHER_WIDTH),
                                             jnp.float32),
               mesh=mesh)
    def gather_kernel(cb_hbm, i_hbm, o_hbm):
        def body(i_vmem, o_vmem):
            pltpu.sync_copy(cb_hbm.at[i_vmem.at[0]], o_vmem)

        pltpu.emit_pipeline(
            body,
            grid=(n_idx // GATHER_WINDOW,),
            in_specs=[pl.BlockSpec((1, GATHER_WINDOW),
                                   index_map=lambda i: (0, i))],
            out_specs=[pl.BlockSpec((GATHER_WINDOW, GATHER_WIDTH),
                                    index_map=lambda i: (i, 0))],
            core_axis_name=("core", "subcore"),
            dimension_semantics=(pltpu.PARALLEL,),
        )(i_hbm, o_hbm)

    return gather_kernel(codebook_padded, indices_2d)


def kernel(z, codebook):
    idx3, loss = _vq_distances_argmin(z, codebook)
    cb_padded = jnp.pad(codebook, ((0, 0), (0, GATHER_WIDTH - EMB_DIM)))
    z_q = _sc_gather(cb_padded, idx3.reshape(1, N_ROWS))
    encoding_indices = idx3.reshape(N_ROWS)
    return (z_q[:, :EMB_DIM].reshape(z.shape), loss.reshape(()),
            encoding_indices)
